# Initial kernel scaffold; baseline (speedup 1.0000x reference)
import jax
import jax.numpy as jnp
from jax.experimental import pallas as pl


def kernel(x, edge_index, edge_mask, edge_attr, We1, be1, We2, be2, Wn1, bn1, Wn2, bn2):
    raise NotImplementedError("write your pallas kernel here")


# speedup vs baseline: 4.6519x; 4.6519x over previous
import functools

import jax
import jax.numpy as jnp
from jax import lax
from jax.experimental import pallas as pl
from jax.experimental.pallas import tpu as pltpu
from jax.experimental.pallas import tpu_sc as plsc

N = 10000
E = 320000
D = 128
DE = 16
H = 128

NC = 2
NS = 16
NW = NC * NS
EPW = E // NW
CB = 80
NCH = EPW // CB
CBS = 40
NCHS = EPW // CBS
RING = 5
WB = 624
WBR = N - NS * WB
ZR = 48

_PREC = jax.lax.Precision.DEFAULT


def _ab_body(x_ref, wa_ref, wb_ref, a_ref, b_ref):
    xv = x_ref[...]
    a_ref[...] = jnp.dot(xv, wa_ref[...], precision=_PREC)
    b_ref[...] = jnp.dot(xv, wb_ref[...], precision=_PREC)


def _edge_body(g_ref, ea_ref, m_ref, w1c_ref, b1_ref, w2_ref, b2_ref, o_ref):
    pre = g_ref[...] + jnp.dot(ea_ref[...], w1c_ref[...], precision=_PREC) + b1_ref[...]
    h = jnp.maximum(pre, 0.0)
    o_ref[...] = (jnp.dot(h, w2_ref[...], precision=_PREC) + b2_ref[...]) * m_ref[...]


def _node_body(x_ref, p_ref, w1x_ref, w1a_ref, b1_ref, w2_ref, b2_ref, o_ref):
    p = p_ref[...]
    agg = p[0] + p[1]
    pre = (jnp.dot(x_ref[...], w1x_ref[...], precision=_PREC)
           + jnp.dot(agg, w1a_ref[...], precision=_PREC) + b1_ref[...])
    h = jnp.maximum(pre, 0.0)
    o_ref[...] = jnp.dot(h, w2_ref[...], precision=_PREC) + b2_ref[...]


def _gather_body(a_hbm, b_hbm, rows_hbm, cols_hbm, g_hbm,
                 idxa, idxb, ba0, bb0, ob0, ba1, bb1, ob1,
                 sa0, sb0, so0, sa1, sb1, so1):
    wid = lax.axis_index("s") * NC + lax.axis_index("c")
    pltpu.sync_copy(rows_hbm.at[wid], idxa)
    pltpu.sync_copy(cols_hbm.at[wid], idxb)
    base = wid * EPW
    sets = ((ba0, bb0, ob0, sa0, sb0, so0),
            (ba1, bb1, ob1, sa1, sb1, so1))

    def issue(c, st):
        ba, bb, _, sa, sb, _ = st
        pltpu.async_copy(a_hbm.at[idxa.at[c]], ba, sa)
        pltpu.async_copy(b_hbm.at[idxb.at[c]], bb, sb)

    def work(c, st, first=False, guard_issue=True):
        ba, bb, ob, sa, sb, so = st
        pltpu.make_async_copy(a_hbm.at[idxa.at[c]], ba, sa).wait()
        pltpu.make_async_copy(b_hbm.at[idxb.at[c]], bb, sb).wait()
        if not first:
            pltpu.make_async_copy(
                ob, g_hbm.at[pl.ds(base + (c - 2) * CB, CB)], so).wait()

        def addrow(r, carry2):
            for l in range(H // 16):
                s = pl.ds(l * 16, 16)
                ob[r, s] = ba[r, s] + bb[r, s]
            return carry2

        lax.fori_loop(0, CB, addrow, 0)
        pltpu.async_copy(ob, g_hbm.at[pl.ds(base + c * CB, CB)], so)
        if guard_issue:
            @pl.when(c + 2 < NCH)
            def _():
                issue(c + 2, st)
        else:
            issue(c + 2, st)

    issue(0, sets[0])
    issue(1, sets[1])
    work(0, sets[0], first=True, guard_issue=False)
    work(1, sets[1], first=True, guard_issue=False)

    def outer(o, carry):
        c = 2 * o + 2
        work(c, sets[0])
        work(c + 1, sets[1])
        return carry

    lax.fori_loop(0, (NCH - 3) // 2, outer, 0)
    work(NCH - 1, sets[0])
    pltpu.make_async_copy(
        ob1, g_hbm.at[pl.ds(base + (NCH - 2) * CB, CB)], so1).wait()
    pltpu.make_async_copy(
        ob0, g_hbm.at[pl.ds(base + (NCH - 1) * CB, CB)], so0).wait()


def _scatter_body(ef_hbm, rows_hbm, out_hbm,
                  b0, b1, b2, b3, b4, i0, i1, i2, i3, i4, zbuf, shared,
                  r0, r1, r2, r3, r4, w0, w1, w2, w3, w4):
    cid = lax.axis_index("c")
    sid = lax.axis_index("s")
    wid = sid * NC + cid

    def zrow(r, carry):
        for l in range(H // 16):
            zbuf[r, pl.ds(l * 16, 16)] = jnp.zeros((16,), jnp.float32)
        return carry

    lax.fori_loop(0, ZR, zrow, 0)
    for j in range(WB // ZR):
        pltpu.sync_copy(zbuf, shared.at[pl.ds(sid * WB + j * ZR, ZR)])

    @pl.when(sid == NS - 1)
    def _zero_tail():
        pltpu.sync_copy(zbuf.at[pl.ds(0, WBR)], shared.at[pl.ds(NS * WB, WBR)])

    plsc.subcore_barrier()

    base = wid * EPW
    bufs = (b0, b1, b2, b3, b4)
    ibufs = (i0, i1, i2, i3, i4)
    rs = (r0, r1, r2, r3, r4)
    ws = (w0, w1, w2, w3, w4)

    def read_issue(c, b):
        pltpu.async_copy(ef_hbm.at[pl.ds(base + c * CBS, CBS)], bufs[b], rs[b])
        pltpu.async_copy(rows_hbm.at[wid, c], ibufs[b], rs[b])

    def read_wait(c, b):
        pltpu.make_async_copy(
            ef_hbm.at[pl.ds(base + c * CBS, CBS)], bufs[b], rs[b]).wait()
        pltpu.make_async_copy(rows_hbm.at[wid, c], ibufs[b], rs[b]).wait()

    for b in range(3):
        read_issue(b, b)

    def step(c, b):
        read_wait(c, b)
        pltpu.async_copy(bufs[b], shared.at[ibufs[b]], ws[b], add=True)
        b3 = (b + 3) % RING

        @pl.when(c >= 2)
        def _drain():
            pltpu.make_async_copy(
                bufs[b3], shared.at[ibufs[b3]], ws[b3]).wait()

        @pl.when(c + 3 < NCHS)
        def _refill():
            read_issue(c + 3, b3)

    def outer(o, carry):
        for b in range(RING):
            step(o * RING + b, b)
        return carry

    lax.fori_loop(0, NCHS // RING, outer, 0)
    pltpu.make_async_copy(
        bufs[3], shared.at[ibufs[3]], ws[3]).wait()
    pltpu.make_async_copy(
        bufs[4], shared.at[ibufs[4]], ws[4]).wait()
    plsc.subcore_barrier()
    pltpu.sync_copy(shared.at[pl.ds(sid * WB, WB)],
                    out_hbm.at[cid, pl.ds(sid * WB, WB)])

    @pl.when(sid == NS - 1)
    def _write_tail():
        pltpu.sync_copy(shared.at[pl.ds(NS * WB, WBR)],
                        out_hbm.at[cid, pl.ds(NS * WB, WBR)])


@functools.cache
def _sc_calls():
    mesh = plsc.VectorSubcoreMesh(core_axis_name="c", subcore_axis_name="s")
    gather_call = pl.kernel(
        _gather_body, mesh=mesh,
        out_type=jax.ShapeDtypeStruct((E, H), jnp.float32),
        scratch_types=[
            pltpu.VMEM((NCH, CB), jnp.int32),
            pltpu.VMEM((NCH, CB), jnp.int32),
        ] + [pltpu.VMEM((CB, H), jnp.float32)] * 6
          + [pltpu.SemaphoreType.DMA] * 6,
    )
    scatter_call = pl.kernel(
        _scatter_body, mesh=mesh,
        out_type=jax.ShapeDtypeStruct((NC, N, H), jnp.float32),
        scratch_types=(
            [pltpu.VMEM((CBS, H), jnp.float32)] * RING
            + [pltpu.VMEM((CBS,), jnp.int32)] * RING
            + [
                pltpu.VMEM((ZR, H), jnp.float32),
                pltpu.VMEM_SHARED((N, H), jnp.float32),
            ] + [pltpu.SemaphoreType.DMA] * (2 * RING)
        ),
    )
    return gather_call, scatter_call


BN = 2000
BE = 3200


def _ab_call(x, wa, wb):
    return pl.pallas_call(
        _ab_body,
        grid=(N // BN,),
        in_specs=[
            pl.BlockSpec((BN, D), lambda i: (i, 0)),
            pl.BlockSpec((D, H), lambda i: (0, 0)),
            pl.BlockSpec((D, H), lambda i: (0, 0)),
        ],
        out_specs=[
            pl.BlockSpec((BN, H), lambda i: (i, 0)),
            pl.BlockSpec((BN, H), lambda i: (i, 0)),
        ],
        out_shape=[
            jax.ShapeDtypeStruct((N, H), jnp.float32),
            jax.ShapeDtypeStruct((N, H), jnp.float32),
        ],
    )(x, wa, wb)


def _edge_call(g, ea, m, w1c, b1, w2, b2):
    return pl.pallas_call(
        _edge_body,
        grid=(E // BE,),
        in_specs=[
            pl.BlockSpec((BE, H), lambda i: (i, 0)),
            pl.BlockSpec((BE, DE), lambda i: (i, 0)),
            pl.BlockSpec((BE, 1), lambda i: (i, 0)),
            pl.BlockSpec((DE, H), lambda i: (0, 0)),
            pl.BlockSpec((1, H), lambda i: (0, 0)),
            pl.BlockSpec((H, H), lambda i: (0, 0)),
            pl.BlockSpec((1, H), lambda i: (0, 0)),
        ],
        out_specs=pl.BlockSpec((BE, H), lambda i: (i, 0)),
        out_shape=jax.ShapeDtypeStruct((E, H), jnp.float32),
    )(g, ea, m, w1c, b1, w2, b2)


def _node_call(x, parts, w1x, w1a, b1, w2, b2):
    return pl.pallas_call(
        _node_body,
        grid=(N // BN,),
        in_specs=[
            pl.BlockSpec((BN, D), lambda i: (i, 0)),
            pl.BlockSpec((NC, BN, H), lambda i: (0, i, 0)),
            pl.BlockSpec((D, H), lambda i: (0, 0)),
            pl.BlockSpec((H, H), lambda i: (0, 0)),
            pl.BlockSpec((1, H), lambda i: (0, 0)),
            pl.BlockSpec((H, D), lambda i: (0, 0)),
            pl.BlockSpec((1, D), lambda i: (0, 0)),
        ],
        out_specs=pl.BlockSpec((BN, D), lambda i: (i, 0)),
        out_shape=jax.ShapeDtypeStruct((N, D), jnp.float32),
    )(x, parts, w1x, w1a, b1, w2, b2)


def kernel(x, edge_index, edge_mask, edge_attr, We1, be1, We2, be2,
           Wn1, bn1, Wn2, bn2):
    row = edge_index[0]
    col = edge_index[1]
    rows3 = row.reshape(NW, NCH, CB)
    cols3 = col.reshape(NW, NCH, CB)
    rows3s = row.reshape(NW, NCHS, CBS)

    gather_call, scatter_call = _sc_calls()
    A, B = _ab_call(x, We1[:D], We1[D:2 * D])
    G = gather_call(A, B, rows3, cols3)
    edge_feat = _edge_call(G, edge_attr, edge_mask, We1[2 * D:],
                           be1.reshape(1, H), We2, be2.reshape(1, H))
    parts = scatter_call(edge_feat, rows3s)
    x_out = _node_call(x, parts, Wn1[:D], Wn1[D:], bn1.reshape(1, H),
                       Wn2, bn2.reshape(1, D))
    return (x_out, edge_feat)
